# Initial kernel scaffold; baseline (speedup 1.0000x reference)
import jax
import jax.numpy as jnp
from jax.experimental import pallas as pl


def kernel(tokens, cu_seqlens, table):
    raise NotImplementedError("write your pallas kernel here")


# speedup vs baseline: 1.3914x; 1.3914x over previous
import functools

import jax
import jax.numpy as jnp
from jax import lax
from jax.experimental import pallas as pl
from jax.experimental.pallas import tpu as pltpu
from jax.experimental.pallas import tpu_sc as plsc

VOCAB = 100000
EMBED = 512
B = 16
MAX_LEN = 4096
PAD_IDX = VOCAB
TOTAL = B * MAX_LEN // 2

LANES = 16
NC, NS = 2, 16
NW = NC * NS
RPW = B * MAX_LEN // NW
CH = 64
NCH = RPW // CH
NPAIR = NCH // 2


def _body(tokens_h, starts_h, lens_h, table_h, out_h,
          tok_v, st_v, ln_v, idx_v, buf_v, gsem0, gsem1):
  core = lax.axis_index("c")
  sub = lax.axis_index("s")
  b = sub
  half = core
  m0 = half * RPW
  row0 = b * MAX_LEN + m0

  pltpu.sync_copy(tokens_h, tok_v)
  pltpu.sync_copy(starts_h, st_v.at[pl.ds(0, LANES)])
  pltpu.sync_copy(lens_h, ln_v.at[pl.ds(0, LANES)])

  lane = jnp.arange(LANES, dtype=jnp.int32)
  start_b = st_v[pl.ds(b, LANES)][0]
  len_b = ln_v[pl.ds(b, LANES)][0]

  def build(i, carry):
    pos = m0 + i * LANES + lane
    valid = pos < len_b
    g = jnp.clip(start_b + pos, 0, TOTAL - 1)
    toks = plsc.load_gather(tok_v, [g])
    idx_v[pl.ds(i * LANES, LANES)] = jnp.where(valid, toks, PAD_IDX)
    return carry

  lax.fori_loop(0, RPW // LANES, build, 0)

  sems = (gsem0, gsem1)

  def issue(ci, j):
    pltpu.async_copy(table_h.at[idx_v.at[pl.ds(ci * CH, CH)]],
                     buf_v.at[j], sems[j])

  def wait(j):
    pltpu.make_async_copy(table_h.at[idx_v.at[pl.ds(0, CH)]],
                          buf_v.at[j], sems[j]).wait()

  def drain(ci, j):
    pltpu.sync_copy(buf_v.at[j], out_h.at[pl.ds(row0 + ci * CH, CH)])

  issue(0, 0)

  def pair(cc, carry):
    ca = cc * 2
    wait(0)
    issue(ca + 1, 1)
    drain(ca, 0)
    wait(1)

    @pl.when(cc < NPAIR - 1)
    def _():
      issue(ca + 2, 0)

    drain(ca + 1, 1)
    return carry

  lax.fori_loop(0, NPAIR, pair, 0)


@jax.jit
def _gather_pallas(tokens, starts, lens, table):
  mesh = plsc.VectorSubcoreMesh(core_axis_name="c", subcore_axis_name="s",
                                num_cores=NC, num_subcores=NS)
  return pl.kernel(
      _body,
      out_type=jax.ShapeDtypeStruct((B * MAX_LEN, EMBED), jnp.float32),
      mesh=mesh,
      compiler_params=pltpu.CompilerParams(needs_layout_passes=False),
      scratch_types=[
          pltpu.VMEM((TOTAL,), jnp.int32),
          pltpu.VMEM((2 * LANES,), jnp.int32),
          pltpu.VMEM((2 * LANES,), jnp.int32),
          pltpu.VMEM((RPW,), jnp.int32),
          pltpu.VMEM((2, CH, EMBED), jnp.float32),
          pltpu.SemaphoreType.DMA,
          pltpu.SemaphoreType.DMA,
      ],
  )(tokens, starts, lens, table)


def kernel(tokens, cu_seqlens, table):
  starts = cu_seqlens[:B]
  lens = cu_seqlens[1:B + 1] - cu_seqlens[:B]
  out = _gather_pallas(tokens, starts, lens, table)
  return out.reshape(B, MAX_LEN, EMBED)
